# Initial kernel scaffold; baseline (speedup 1.0000x reference)
import jax
import jax.numpy as jnp
from jax.experimental import pallas as pl


def kernel(x, seq_lengths):
    raise NotImplementedError("write your pallas kernel here")


# speedup vs baseline: 1.6950x; 1.6950x over previous
import functools

import jax
import jax.numpy as jnp
from jax import lax
from jax.experimental import pallas as pl
from jax.experimental.pallas import tpu as pltpu
from jax.experimental.pallas import tpu_sc as plsc

B, S, D = 16, 4096, 1024
NC, NS, LANES = 2, 16, 16
NW = NC * NS
ROWS_PER_W = (B * S) // NW
CH = 64
NCHUNK = ROWS_PER_W // CH


def _make_sc_reverse():
    mesh = plsc.VectorSubcoreMesh(core_axis_name="c", subcore_axis_name="s")

    @functools.partial(
        pl.kernel,
        mesh=mesh,
        out_type=jax.ShapeDtypeStruct((B * S, D), jnp.float32),
        scratch_types=[
            pltpu.VMEM((B * LANES,), jnp.int32),
            pltpu.VMEM((ROWS_PER_W,), jnp.int32),
            pltpu.VMEM((CH, D), jnp.float32),
            pltpu.SemaphoreType.DMA,
        ],
    )
    def k(x_hbm, len_hbm, out_hbm, len_v, idx_v, buf, sem):
        wid = lax.axis_index("s") * NC + lax.axis_index("c")
        b = wid // 2
        s_base = (wid % 2) * ROWS_PER_W
        row0 = b * S

        pltpu.sync_copy(len_hbm, len_v)
        lvec = len_v[pl.ds(b * LANES, LANES)][0]

        def build(kk, _):
            p = s_base + kk * LANES + lax.iota(jnp.int32, LANES)
            src = jnp.where(p < lvec, lvec - 1 - p, p) + row0
            idx_v[pl.ds(kk * LANES, LANES)] = src
            return 0

        lax.fori_loop(0, ROWS_PER_W // LANES, build, 0)

        def chunk(c, _):
            pltpu.async_copy(x_hbm.at[idx_v.at[pl.ds(c * CH, CH)]], buf, sem).wait()
            pltpu.sync_copy(buf, out_hbm.at[pl.ds(row0 + s_base + c * CH, CH)])
            return 0

        lax.fori_loop(0, NCHUNK, chunk, 0)

    return k


def kernel(x, seq_lengths):
    x2 = x.reshape(B * S, D)
    lens = jnp.repeat(seq_lengths.astype(jnp.int32), LANES)
    out = _make_sc_reverse()(x2, lens)
    return out.reshape(B, S, D)


# speedup vs baseline: 1.8150x; 1.0708x over previous
import functools

import jax
import jax.numpy as jnp
from jax import lax
from jax.experimental import pallas as pl
from jax.experimental.pallas import tpu as pltpu
from jax.experimental.pallas import tpu_sc as plsc

B, S, D = 16, 4096, 1024
NC, NS, LANES = 2, 16, 16
NW = NC * NS
ROWS_PER_W = (B * S) // NW
CH = 32
NCHUNK = ROWS_PER_W // CH


def _make_sc_reverse():
    mesh = plsc.VectorSubcoreMesh(core_axis_name="c", subcore_axis_name="s")

    @functools.partial(
        pl.kernel,
        mesh=mesh,
        out_type=jax.ShapeDtypeStruct((B * S, D), jnp.float32),
        scratch_types=[
            pltpu.VMEM((B * LANES,), jnp.int32),
            pltpu.VMEM((ROWS_PER_W,), jnp.int32),
            pltpu.VMEM((CH, D), jnp.float32),
            pltpu.VMEM((CH, D), jnp.float32),
            pltpu.SemaphoreType.DMA,
            pltpu.SemaphoreType.DMA,
            pltpu.SemaphoreType.DMA,
            pltpu.SemaphoreType.DMA,
        ],
    )
    def k(x_hbm, len_hbm, out_hbm, len_v, idx_v, buf0, buf1, sg0, sg1, ss0, ss1):
        wid = lax.axis_index("s") * NC + lax.axis_index("c")
        b = wid // 2
        s_base = (wid % 2) * ROWS_PER_W
        row0 = b * S

        pltpu.sync_copy(len_hbm, len_v)
        lvec = len_v[pl.ds(b * LANES, LANES)][0]

        def build(kk, _):
            p = s_base + kk * LANES + lax.iota(jnp.int32, LANES)
            src = jnp.where(p < lvec, lvec - 1 - p, p) + row0
            idx_v[pl.ds(kk * LANES, LANES)] = src
            return 0

        lax.fori_loop(0, ROWS_PER_W // LANES, build, 0)

        out0 = row0 + s_base
        bufs = (buf0, buf1)
        g_sems = (sg0, sg1)
        s_sems = (ss0, ss1)

        def gather(c, j):
            return pltpu.make_async_copy(
                x_hbm.at[idx_v.at[pl.ds(c * CH, CH)]], bufs[j], g_sems[j])

        def scatter(c, j):
            return pltpu.make_async_copy(
                bufs[j], out_hbm.at[pl.ds(out0 + c * CH, CH)], s_sems[j])

        def step(c, j):
            gather(c, j).wait()
            scatter(c, j).start()

            @pl.when(c >= 1)
            def _():
                scatter(c - 1, 1 - j).wait()

            @pl.when(c + 1 < NCHUNK)
            def _():
                gather(c + 1, 1 - j).start()

        gather(0, 0).start()

        def pair(i, _):
            step(2 * i, 0)
            step(2 * i + 1, 1)
            return 0

        lax.fori_loop(0, NCHUNK // 2, pair, 0)
        scatter(NCHUNK - 1, (NCHUNK - 1) % 2).wait()

    return k


def kernel(x, seq_lengths):
    x2 = x.reshape(B * S, D)
    lens = jnp.repeat(seq_lengths.astype(jnp.int32), LANES)
    out = _make_sc_reverse()(x2, lens)
    return out.reshape(B, S, D)
